# baseline (speedup 1.0000x reference)
import jax
import jax.numpy as jnp
from jax import lax
from jax.experimental import pallas as pl
from jax.experimental.pallas import tpu as pltpu
from jax.experimental.pallas import tpu_sc as plsc

D_MODEL = 1024
SEQ_LEN = 2048
BATCH = 4
SCALE = 32.0

NC, NS, L = 2, 16, 16
NW = NC * NS
NROWS = SEQ_LEN * BATCH
ROWS_PER_W = NROWS // NW
CHUNK = 16
NCHUNK = ROWS_PER_W // CHUNK
PE_PER_W = ROWS_PER_W // BATCH


def _body(idx_hbm, pe_hbm, table_hbm, out_hbm, idx_v, pe_v, buf, gsem):
    wid = lax.axis_index("s") * NC + lax.axis_index("c")
    base = wid * ROWS_PER_W
    pltpu.sync_copy(idx_hbm.at[pl.ds(base, ROWS_PER_W)], idx_v)
    pltpu.sync_copy(pe_hbm.at[pl.ds(wid * PE_PER_W, PE_PER_W)], pe_v)

    def chunk_body(c, carry):
        off = pl.multiple_of(c * CHUNK, CHUNK)
        pltpu.async_copy(table_hbm.at[idx_v.at[pl.ds(off, CHUNK)]], buf, gsem).wait()
        for g in range(CHUNK // BATCH):
            pe_row = c * (CHUNK // BATCH) + g

            def jbody(j, inner):
                col = pl.ds(pl.multiple_of(j * L, L), L)
                pv = pe_v[pe_row, col]
                for r in range(BATCH):
                    row = g * BATCH + r
                    buf[row, col] = buf[row, col] * SCALE + pv
                return inner

            lax.fori_loop(0, D_MODEL // L, jbody, 0)
        pltpu.sync_copy(buf, out_hbm.at[pl.ds(base + off, CHUNK)])
        return carry

    lax.fori_loop(0, NCHUNK, chunk_body, 0)


_mesh = plsc.VectorSubcoreMesh(core_axis_name="c", subcore_axis_name="s")

_emb = pl.kernel(
    _body,
    mesh=_mesh,
    out_type=jax.ShapeDtypeStruct((NROWS, D_MODEL), jnp.float32),
    scratch_types=[
        pltpu.VMEM((ROWS_PER_W,), jnp.int32),
        pltpu.VMEM((PE_PER_W, D_MODEL), jnp.float32),
        pltpu.VMEM((CHUNK, D_MODEL), jnp.float32),
        pltpu.SemaphoreType.DMA,
    ],
)


def kernel(x, table, pe):
    idx = x.reshape(-1).astype(jnp.int32)
    pe2d = pe[: x.shape[0], 0, :]
    out = _emb(idx, pe2d, table)
    return out.reshape(x.shape[0], x.shape[1], D_MODEL)
